# baseline (speedup 1.0000x reference)
import functools

import jax
import jax.numpy as jnp
from jax import lax
from jax.experimental import pallas as pl
from jax.experimental.pallas import tpu as pltpu
from jax.experimental.pallas import tpu_sc as plsc

ENT_SIZE = 1000000
DIM = 32
NUM_CHUNK = 16
POS_NUM = 1024
NEG_NUM = 1024
PN = POS_NUM + NEG_NUM
PACK = 4
_T_BLK = 32768
_SUB = _T_BLK // PACK
_N_T_BLK = (ENT_SIZE + _T_BLK - 1) // _T_BLK
N_PACKED = _N_T_BLK * _SUB
_LOG_T = _T_BLK.bit_length() - 1
_LOG_S = _SUB.bit_length() - 1

_IDX_ROWS = 4 * NUM_CHUNK * POS_NUM // 128
_NW = 32
_ROWS_PER_W = _IDX_ROWS // _NW


def _tc_repack_body(t_ref, out_ref):
    x = t_ref[...]
    for a in range(PACK):
        out_ref[:, a * DIM:(a + 1) * DIM] = jnp.transpose(
            x[:, a * _SUB:(a + 1) * _SUB], (1, 0))


_tc_repack = pl.pallas_call(
    _tc_repack_body,
    grid=(_N_T_BLK,),
    in_specs=[pl.BlockSpec((DIM, _T_BLK), lambda i: (0, i))],
    out_specs=pl.BlockSpec((_SUB, PACK * DIM), lambda i: (i, 0)),
    out_shape=jax.ShapeDtypeStruct((N_PACKED, PACK * DIM), jnp.float32),
)


_HALF_ROWS = _IDX_ROWS // 4
_HROWS_PER_W = _HALF_ROWS // _NW


@functools.cache
def _make_sc_gather():
    @functools.partial(
        pl.kernel,
        mesh=plsc.VectorSubcoreMesh(core_axis_name="c", subcore_axis_name="s"),
        out_type=jax.ShapeDtypeStruct((_HALF_ROWS, 128, 128), jnp.float32),
        scratch_types=[
            pltpu.VMEM((_HROWS_PER_W, 128), jnp.int32),
            pltpu.VMEM((2, 128, 128), jnp.float32),
            pltpu.SemaphoreType.DMA,
            pltpu.SemaphoreType.DMA,
        ],
    )
    def _sc_gather(tab_hbm, idx_hbm, out_hbm, idx_v, rows_v, sem0, sem1):
        wid = lax.axis_index("s") * 2 + lax.axis_index("c")
        base = wid * _HROWS_PER_W
        pltpu.sync_copy(idx_hbm.at[pl.ds(base, _HROWS_PER_W)], idx_v)
        sems = (sem0, sem1)
        copies = [None, None]
        copies[0] = pltpu.async_copy(tab_hbm.at[idx_v.at[0]], rows_v.at[0], sem0)
        for j in range(_HROWS_PER_W):
            b = j % 2
            if j + 1 < _HROWS_PER_W:
                copies[(j + 1) % 2] = pltpu.async_copy(
                    tab_hbm.at[idx_v.at[j + 1]], rows_v.at[(j + 1) % 2],
                    sems[(j + 1) % 2],
                )
            copies[b].wait()
            pltpu.sync_copy(rows_v.at[b], out_hbm.at[base + j])

    return _sc_gather


def _tc_score_body(g_ref, q_ref, pos_ref, hn_ref, tn_ref):
    def sel(t):
        g = g_ref[0, t]
        qv = q_ref[0, t]
        out = g[:, 0:DIM]
        for qq in range(1, PACK):
            out = jnp.where(qv == qq, g[:, qq * DIM:(qq + 1) * DIM], out)
        return out

    head = sel(0)
    tail = sel(1)
    hne = sel(2)
    tne = sel(3)
    dn = (((1,), (1,)), ((), ()))
    s_ht = lax.dot_general(head, tail, dn, preferred_element_type=jnp.float32)
    s_hn = lax.dot_general(head, tne, dn, preferred_element_type=jnp.float32)
    s_th = lax.dot_general(tail, head, dn, preferred_element_type=jnp.float32)
    s_tn = lax.dot_general(tail, hne, dn, preferred_element_type=jnp.float32)
    pos_f = jnp.sum(head * tail, axis=1, keepdims=True)
    rows = lax.broadcasted_iota(jnp.int32, (POS_NUM, POS_NUM), 0)
    cols = lax.broadcasted_iota(jnp.int32, (POS_NUM, POS_NUM), 1)
    neg = jnp.where(rows == cols, jnp.float32(-1000000000.0), jnp.float32(0.0))
    hn_ref[0, :, 0:POS_NUM] = s_ht + neg
    hn_ref[0, :, POS_NUM:PN] = s_hn
    tn_ref[0, :, 0:POS_NUM] = s_th + neg
    tn_ref[0, :, POS_NUM:PN] = s_tn
    pos_ref[0] = pos_f


_QC = NUM_CHUNK // 4

_OUT_SHAPES = [
    jax.ShapeDtypeStruct((NUM_CHUNK, POS_NUM, 1), jnp.float32),
    jax.ShapeDtypeStruct((NUM_CHUNK, POS_NUM, PN), jnp.float32),
    jax.ShapeDtypeStruct((NUM_CHUNK, POS_NUM, PN), jnp.float32),
]

_IN_SPECS = [
    pl.BlockSpec((1, 4, POS_NUM, 128), lambda c: (c, 0, 0, 0)),
    pl.BlockSpec((1, 4, POS_NUM, 1), lambda c: (c, 0, 0, 0)),
]


def _tc_score_chain_body(g_ref, q_ref, p0, p1, p2, pos_ref, hn_ref, tn_ref):
    del p0, p1, p2
    _tc_score_body(g_ref, q_ref, pos_ref, hn_ref, tn_ref)


def _make_score(off):
    def out_specs():
        return [
            pl.BlockSpec((1, POS_NUM, 1), lambda c: (c + off, 0, 0)),
            pl.BlockSpec((1, POS_NUM, PN), lambda c: (c + off, 0, 0)),
            pl.BlockSpec((1, POS_NUM, PN), lambda c: (c + off, 0, 0)),
        ]

    if off == 0:
        return pl.pallas_call(
            _tc_score_body,
            grid=(_QC,),
            in_specs=_IN_SPECS,
            out_specs=out_specs(),
            out_shape=_OUT_SHAPES,
        )
    return pl.pallas_call(
        _tc_score_chain_body,
        grid=(_QC,),
        in_specs=_IN_SPECS + [
            pl.BlockSpec(memory_space=pltpu.MemorySpace.HBM),
            pl.BlockSpec(memory_space=pltpu.MemorySpace.HBM),
            pl.BlockSpec(memory_space=pltpu.MemorySpace.HBM),
        ],
        out_specs=out_specs(),
        out_shape=_OUT_SHAPES,
        input_output_aliases={2: 0, 3: 1, 4: 2},
    )


_tc_scores = [_make_score(off) for off in range(0, NUM_CHUNK, _QC)]


def kernel(head_index, tail_index, head_neg_index, tail_neg_index, rel_index, emb):
    del rel_index
    idx = jnp.stack(
        [head_index, tail_index, head_neg_index, tail_neg_index]
    ).astype(jnp.int32).transpose(1, 0, 2)
    row = (idx >> _LOG_T) * _SUB + (idx & (_SUB - 1))
    idx_grid = row.reshape(_IDX_ROWS, 128)
    q = ((idx >> _LOG_S) & 3).reshape(NUM_CHUNK, 4, POS_NUM, 1)
    emb_t = emb.T
    tab = _tc_repack(emb_t)
    gather = _make_sc_gather()
    gs = [
        gather(tab, idx_grid[i * _HALF_ROWS:(i + 1) * _HALF_ROWS])
        .reshape(_QC, 4, POS_NUM, 128)
        for i in range(4)
    ]
    outs = _tc_scores[0](gs[0], q[:_QC])
    for i in range(1, 4):
        outs = _tc_scores[i](gs[i], q[i * _QC:(i + 1) * _QC], *outs)
    pos, hn, tn = outs
    pos2 = pos.reshape(NUM_CHUNK * POS_NUM, 1)
    return (
        pos2,
        pos2,
        hn.reshape(NUM_CHUNK * POS_NUM, PN),
        tn.reshape(NUM_CHUNK * POS_NUM, PN),
    )


# speedup vs baseline: 1.0295x; 1.0295x over previous
import functools

import jax
import jax.numpy as jnp
from jax import lax
from jax.experimental import pallas as pl
from jax.experimental.pallas import tpu as pltpu
from jax.experimental.pallas import tpu_sc as plsc

ENT_SIZE = 1000000
DIM = 32
NUM_CHUNK = 16
POS_NUM = 1024
NEG_NUM = 1024
PN = POS_NUM + NEG_NUM
PACK = 4
_T_BLK = 32768
_SUB = _T_BLK // PACK
_N_T_BLK = (ENT_SIZE + _T_BLK - 1) // _T_BLK
N_PACKED = _N_T_BLK * _SUB
_LOG_T = _T_BLK.bit_length() - 1
_LOG_S = _SUB.bit_length() - 1

_IDX_ROWS = 4 * NUM_CHUNK * POS_NUM // 128
_NW = 32
_ROWS_PER_W = _IDX_ROWS // _NW


def _tc_repack_body(t_ref, out_ref):
    x = t_ref[...]
    for a in range(PACK):
        out_ref[:, a * DIM:(a + 1) * DIM] = jnp.transpose(
            x[:, a * _SUB:(a + 1) * _SUB], (1, 0))


_tc_repack = pl.pallas_call(
    _tc_repack_body,
    grid=(_N_T_BLK,),
    in_specs=[pl.BlockSpec((DIM, _T_BLK), lambda i: (0, i))],
    out_specs=pl.BlockSpec((_SUB, PACK * DIM), lambda i: (i, 0)),
    out_shape=jax.ShapeDtypeStruct((N_PACKED, PACK * DIM), jnp.float32),
)


_HALF_ROWS = _IDX_ROWS // 2
_HROWS_PER_W = _HALF_ROWS // _NW


@functools.cache
def _make_sc_gather():
    @functools.partial(
        pl.kernel,
        mesh=plsc.VectorSubcoreMesh(core_axis_name="c", subcore_axis_name="s"),
        out_type=jax.ShapeDtypeStruct((_HALF_ROWS, 128, 128), jnp.float32),
        scratch_types=[
            pltpu.VMEM((_HROWS_PER_W, 128), jnp.int32),
            pltpu.VMEM((2, 128, 128), jnp.float32),
            pltpu.SemaphoreType.DMA,
            pltpu.SemaphoreType.DMA,
        ],
    )
    def _sc_gather(tab_hbm, idx_hbm, out_hbm, idx_v, rows_v, sem0, sem1):
        wid = lax.axis_index("s") * 2 + lax.axis_index("c")
        base = wid * _HROWS_PER_W
        pltpu.sync_copy(idx_hbm.at[pl.ds(base, _HROWS_PER_W)], idx_v)
        sems = (sem0, sem1)
        copies = [None, None]
        copies[0] = pltpu.async_copy(tab_hbm.at[idx_v.at[0]], rows_v.at[0], sem0)
        for j in range(_HROWS_PER_W):
            b = j % 2
            if j + 1 < _HROWS_PER_W:
                copies[(j + 1) % 2] = pltpu.async_copy(
                    tab_hbm.at[idx_v.at[j + 1]], rows_v.at[(j + 1) % 2],
                    sems[(j + 1) % 2],
                )
            copies[b].wait()
            pltpu.sync_copy(rows_v.at[b], out_hbm.at[base + j])

    return _sc_gather


def _tc_score_body(g_ref, q_ref, pos_ref, hn_ref, tn_ref):
    def sel(t):
        g = g_ref[0, t]
        qv = q_ref[0, t]
        out = g[:, 0:DIM]
        for qq in range(1, PACK):
            out = jnp.where(qv == qq, g[:, qq * DIM:(qq + 1) * DIM], out)
        return out

    head = sel(0)
    tail = sel(1)
    hne = sel(2)
    tne = sel(3)
    dn = (((1,), (1,)), ((), ()))
    s_ht = lax.dot_general(head, tail, dn, preferred_element_type=jnp.float32)
    s_hn = lax.dot_general(head, tne, dn, preferred_element_type=jnp.float32)
    s_th = lax.dot_general(tail, head, dn, preferred_element_type=jnp.float32)
    s_tn = lax.dot_general(tail, hne, dn, preferred_element_type=jnp.float32)
    pos_f = jnp.sum(head * tail, axis=1, keepdims=True)
    rows = lax.broadcasted_iota(jnp.int32, (POS_NUM, POS_NUM), 0)
    cols = lax.broadcasted_iota(jnp.int32, (POS_NUM, POS_NUM), 1)
    neg = jnp.where(rows == cols, jnp.float32(-1000000000.0), jnp.float32(0.0))
    hn_ref[0, :, 0:POS_NUM] = s_ht + neg
    hn_ref[0, :, POS_NUM:PN] = s_hn
    tn_ref[0, :, 0:POS_NUM] = s_th + neg
    tn_ref[0, :, POS_NUM:PN] = s_tn
    pos_ref[0] = pos_f


_HC = NUM_CHUNK // 2

_OUT_SHAPES = [
    jax.ShapeDtypeStruct((NUM_CHUNK, POS_NUM, 1), jnp.float32),
    jax.ShapeDtypeStruct((NUM_CHUNK, POS_NUM, PN), jnp.float32),
    jax.ShapeDtypeStruct((NUM_CHUNK, POS_NUM, PN), jnp.float32),
]

_OUT_SPECS = [
    pl.BlockSpec((1, POS_NUM, 1), lambda c: (c, 0, 0)),
    pl.BlockSpec((1, POS_NUM, PN), lambda c: (c, 0, 0)),
    pl.BlockSpec((1, POS_NUM, PN), lambda c: (c, 0, 0)),
]

_OUT_SPECS_HI = [
    pl.BlockSpec((1, POS_NUM, 1), lambda c: (c + _HC, 0, 0)),
    pl.BlockSpec((1, POS_NUM, PN), lambda c: (c + _HC, 0, 0)),
    pl.BlockSpec((1, POS_NUM, PN), lambda c: (c + _HC, 0, 0)),
]

_IN_SPECS = [
    pl.BlockSpec((1, 4, POS_NUM, 128), lambda c: (c, 0, 0, 0)),
    pl.BlockSpec((1, 4, POS_NUM, 1), lambda c: (c, 0, 0, 0)),
]

_tc_score_lo = pl.pallas_call(
    _tc_score_body,
    grid=(_HC,),
    in_specs=_IN_SPECS,
    out_specs=_OUT_SPECS,
    out_shape=_OUT_SHAPES,
)


def _tc_score_hi_body(g_ref, q_ref, p0, p1, p2, pos_ref, hn_ref, tn_ref):
    del p0, p1, p2
    _tc_score_body(g_ref, q_ref, pos_ref, hn_ref, tn_ref)


_tc_score_hi = pl.pallas_call(
    _tc_score_hi_body,
    grid=(_HC,),
    in_specs=_IN_SPECS + [
        pl.BlockSpec(memory_space=pltpu.MemorySpace.HBM),
        pl.BlockSpec(memory_space=pltpu.MemorySpace.HBM),
        pl.BlockSpec(memory_space=pltpu.MemorySpace.HBM),
    ],
    out_specs=_OUT_SPECS_HI,
    out_shape=_OUT_SHAPES,
    input_output_aliases={2: 0, 3: 1, 4: 2},
)


def kernel(head_index, tail_index, head_neg_index, tail_neg_index, rel_index, emb):
    del rel_index
    idx = jnp.stack(
        [head_index, tail_index, head_neg_index, tail_neg_index]
    ).astype(jnp.int32).transpose(1, 0, 2)
    row = (idx >> _LOG_T) * _SUB + (idx & (_SUB - 1))
    idx_grid = row.reshape(_IDX_ROWS, 128)
    q = ((idx >> _LOG_S) & 3).reshape(NUM_CHUNK, 4, POS_NUM, 1)
    emb_t = emb.T
    tab = _tc_repack(emb_t)
    gather = _make_sc_gather()
    g_lo = gather(tab, idx_grid[:_HALF_ROWS])
    g_hi = gather(tab, idx_grid[_HALF_ROWS:])
    g_lo = g_lo.reshape(_HC, 4, POS_NUM, 128)
    g_hi = g_hi.reshape(_HC, 4, POS_NUM, 128)
    pos, hn, tn = _tc_score_lo(g_lo, q[:_HC])
    pos, hn, tn = _tc_score_hi(g_hi, q[_HC:], pos, hn, tn)
    pos2 = pos.reshape(NUM_CHUNK * POS_NUM, 1)
    return (
        pos2,
        pos2,
        hn.reshape(NUM_CHUNK * POS_NUM, PN),
        tn.reshape(NUM_CHUNK * POS_NUM, PN),
    )
